# baseline (speedup 1.0000x reference)
import functools
import jax
import jax.numpy as jnp
from jax import lax
from jax.experimental import pallas as pl
from jax.experimental.pallas import tpu as pltpu
from jax.experimental.pallas import tpu_sc as plsc

_WIN = 16


def _mr_aggregate_sc(x_rows, i_src, i_dst, K):
    _, C = x_rows.shape
    nblk = i_src.shape[0]
    S_pad = nblk * _WIN
    NV = C // 16
    GR = _WIN * K
    NI = GR // 128
    mesh = plsc.VectorSubcoreMesh(
        core_axis_name="c", subcore_axis_name="s", num_cores=2, num_subcores=16
    )

    @functools.partial(
        pl.kernel,
        out_type=jax.ShapeDtypeStruct((S_pad, C), jnp.float32),
        mesh=mesh,
        scratch_types=[
            pltpu.VMEM((GR, C), jnp.float32),
            pltpu.VMEM((GR, C), jnp.float32),
            pltpu.SemaphoreType.DMA,
            pltpu.SemaphoreType.DMA,
        ],
    )
    def sc_kernel(x_hbm, isrc_hbm, idst_hbm, o_hbm, xj_v, xi_v, semj, semi):
        def body(isrc_vmem, idst_vmem, o_vmem):
            copies = []
            for q in range(NI):
                sl = pl.ds(q * 128, 128)
                copies.append(pltpu.async_copy(
                    x_hbm.at[isrc_vmem.at[0, 0, sl]],
                    xj_v.at[pl.ds(q * 128, 128)], semj))
                copies.append(pltpu.async_copy(
                    x_hbm.at[idst_vmem.at[0, 0, sl]],
                    xi_v.at[pl.ds(q * 128, 128)], semi))
            for cp in copies:
                cp.wait()

            @pl.loop(0, _WIN)
            def _(m):
                rb = m * K
                for v in range(NV):
                    sl = pl.ds(v * 16, 16)
                    acc = xj_v[rb, sl] - xi_v[rb, sl]
                    for kk in range(1, K):
                        acc = jnp.maximum(
                            acc, xj_v[rb + kk, sl] - xi_v[rb + kk, sl]
                        )
                    o_vmem[m, sl] = acc

        pltpu.emit_pipeline(
            body,
            grid=(nblk,),
            in_specs=[
                pl.BlockSpec((1, 1, GR), lambda i: (i, 0, 0)),
                pl.BlockSpec((1, 1, GR), lambda i: (i, 0, 0)),
            ],
            out_specs=[pl.BlockSpec((_WIN, C), lambda i: (i, 0))],
            core_axis_name=("c", "s"),
            dimension_semantics=(pltpu.PARALLEL,),
        )(isrc_hbm, idst_hbm, o_hbm)

    return sc_kernel(x_rows, i_src, i_dst)


def _to_rows_tc(xt):
    B, C, N = xt.shape

    def body(x_ref, o_ref):
        o_ref[0] = jnp.transpose(x_ref[0], (1, 0))

    rows = pl.pallas_call(
        body,
        grid=(B,),
        in_specs=[pl.BlockSpec((1, C, N), lambda b: (b, 0, 0))],
        out_specs=pl.BlockSpec((1, N, C), lambda b: (b, 0, 0)),
        out_shape=jax.ShapeDtypeStruct((B, N, C), jnp.float32),
    )(xt)
    return rows.reshape(B * N, C)


def _conv_stats_tc(xt, xm_rows, Wg, b2, BN):
    B, C, N = xt.shape
    G, opg, cpg = Wg.shape
    out_ch = G * opg
    NB = N // BN

    def body(x_ref, xm_ref, w_ref, b_ref, y_ref, s_ref):
        step = pl.program_id(0) * pl.num_programs(1) + pl.program_id(1)
        xblk = x_ref[0]
        xmblk = xm_ref[...]
        parts = []
        for g in range(G):
            w = w_ref[g]
            if g * cpg < C:
                h = xblk[g * cpg:(g + 1) * cpg, :]
                yg = lax.dot_general(
                    w, h, (((1,), (0,)), ((), ())),
                    preferred_element_type=jnp.float32,
                    precision=lax.Precision.HIGHEST,
                )
            else:
                c0 = g * cpg - C
                h = xmblk[:, c0:c0 + cpg]
                yg = lax.dot_general(
                    w, h, (((1,), (1,)), ((), ())),
                    preferred_element_type=jnp.float32,
                    precision=lax.Precision.HIGHEST,
                )
            parts.append(yg)
        y = jnp.concatenate(parts, axis=0) + b_ref[...]
        y_ref[0] = y

        @pl.when(step == 0)
        def _():
            s_ref[...] = jnp.zeros_like(s_ref)

        s_ref[:, 0:1] += jnp.sum(y, axis=1, keepdims=True)
        s_ref[:, 1:2] += jnp.sum(y * y, axis=1, keepdims=True)

    return pl.pallas_call(
        body,
        grid=(B, NB),
        in_specs=[
            pl.BlockSpec((1, C, BN), lambda bi, ni: (bi, 0, ni)),
            pl.BlockSpec((BN, C), lambda bi, ni: (bi * (N // BN) + ni, 0)),
            pl.BlockSpec((G, opg, cpg), lambda bi, ni: (0, 0, 0)),
            pl.BlockSpec((out_ch, 1), lambda bi, ni: (0, 0)),
        ],
        out_specs=[
            pl.BlockSpec((1, out_ch, BN), lambda bi, ni: (bi, 0, ni)),
            pl.BlockSpec((out_ch, 2), lambda bi, ni: (0, 0)),
        ],
        out_shape=[
            jax.ShapeDtypeStruct((B, out_ch, N), jnp.float32),
            jax.ShapeDtypeStruct((out_ch, 2), jnp.float32),
        ],
    )(xt, xm_rows, Wg, b2)


def _bn_gelu_tc(y, sums, g2, be2, BN):
    B, out_ch, N = y.shape
    NB = N // BN
    count = float(B * N)

    def body(y_ref, s_ref, g_ref, be_ref, o_ref):
        mean = s_ref[:, 0:1] * (1.0 / count)
        var = s_ref[:, 1:2] * (1.0 / count) - mean * mean
        inv = lax.rsqrt(var + 1e-5)
        scale = g_ref[...] * inv
        shift = be_ref[...] - mean * scale
        z = y_ref[0] * scale + shift
        o_ref[0] = 0.5 * z * (1.0 + lax.erf(z * 0.7071067811865475))

    return pl.pallas_call(
        body,
        grid=(B, NB),
        in_specs=[
            pl.BlockSpec((1, out_ch, BN), lambda bi, ni: (bi, 0, ni)),
            pl.BlockSpec((out_ch, 2), lambda bi, ni: (0, 0)),
            pl.BlockSpec((out_ch, 1), lambda bi, ni: (0, 0)),
            pl.BlockSpec((out_ch, 1), lambda bi, ni: (0, 0)),
        ],
        out_specs=pl.BlockSpec((1, out_ch, BN), lambda bi, ni: (bi, 0, ni)),
        out_shape=jax.ShapeDtypeStruct((B, out_ch, N), jnp.float32),
    )(y, sums, g2, be2)


@jax.jit
def kernel(x, edge_index, W, b, gamma, beta):
    B, C, N, _ = x.shape
    K = edge_index.shape[-1]
    S = B * N
    out_ch = W.shape[0]
    G = 4
    cpg = (2 * C) // G
    opg = out_ch // G

    xt = x[:, :, :, 0]
    x_rows = _to_rows_tc(xt)
    offs = (jnp.arange(B, dtype=jnp.int32) * N)[None, :, None, None]
    eg = edge_index + offs

    S_pad = ((S + 32 * _WIN - 1) // (32 * _WIN)) * (32 * _WIN)
    npad = S_pad - S
    pad = jnp.zeros((npad * K,), dtype=jnp.int32)
    i_src = jnp.concatenate([eg[0].reshape(S * K), pad]).reshape(-1, 1, _WIN * K)
    i_dst = jnp.concatenate([eg[1].reshape(S * K), pad]).reshape(-1, 1, _WIN * K)

    xmax_rows = _mr_aggregate_sc(x_rows, i_src, i_dst, K)

    Wg = W[:, :, 0, 0].reshape(G, opg, cpg)
    BN = N
    y, sums = _conv_stats_tc(xt, xmax_rows, Wg, b.reshape(out_ch, 1), BN)
    out = _bn_gelu_tc(y, sums, gamma.reshape(out_ch, 1),
                      beta.reshape(out_ch, 1), BN)
    return out.reshape(B, out_ch, N, 1)


# speedup vs baseline: 1.5578x; 1.5578x over previous
import functools
import jax
import jax.numpy as jnp
from jax import lax
from jax.experimental import pallas as pl
from jax.experimental.pallas import tpu as pltpu
from jax.experimental.pallas import tpu_sc as plsc

_WIN = 8


def _mr_aggregate_sc(x_rows, i_src, i_dst, K):
    _, C = x_rows.shape
    nblk = i_src.shape[0]
    S_pad = nblk * _WIN
    NV = C // 16
    GR = _WIN * K
    NI = GR // 128
    mesh = plsc.VectorSubcoreMesh(
        core_axis_name="c", subcore_axis_name="s", num_cores=2, num_subcores=16
    )

    @functools.partial(
        pl.kernel,
        out_type=jax.ShapeDtypeStruct((S_pad, C), jnp.float32),
        mesh=mesh,
        scratch_types=[
            pltpu.VMEM((GR, C), jnp.float32),
            pltpu.VMEM((GR, C), jnp.float32),
            pltpu.SemaphoreType.DMA,
            pltpu.SemaphoreType.DMA,
        ],
    )
    def sc_kernel(x_hbm, isrc_hbm, idst_hbm, o_hbm, xj_v, xi_v, semj, semi):
        def body(isrc_vmem, idst_vmem, o_vmem):
            copies = []
            for q in range(NI):
                sl = pl.ds(q * 128, 128)
                copies.append(pltpu.async_copy(
                    x_hbm.at[isrc_vmem.at[0, 0, sl]],
                    xj_v.at[pl.ds(q * 128, 128)], semj))
                copies.append(pltpu.async_copy(
                    x_hbm.at[idst_vmem.at[0, 0, sl]],
                    xi_v.at[pl.ds(q * 128, 128)], semi))
            for cp in copies:
                cp.wait()

            @pl.loop(0, _WIN)
            def _(m):
                rb = m * K
                for v in range(NV):
                    sl = pl.ds(v * 16, 16)
                    acc = xj_v[rb, sl] - xi_v[rb, sl]
                    for kk in range(1, K):
                        acc = jnp.maximum(
                            acc, xj_v[rb + kk, sl] - xi_v[rb + kk, sl]
                        )
                    o_vmem[m, sl] = acc

        pltpu.emit_pipeline(
            body,
            grid=(nblk,),
            in_specs=[
                pl.BlockSpec((1, 1, GR), lambda i: (i, 0, 0)),
                pl.BlockSpec((1, 1, GR), lambda i: (i, 0, 0)),
            ],
            out_specs=[pl.BlockSpec((_WIN, C), lambda i: (i, 0))],
            core_axis_name=("c", "s"),
            dimension_semantics=(pltpu.PARALLEL,),
        )(isrc_hbm, idst_hbm, o_hbm)

    return sc_kernel(x_rows, i_src, i_dst)


def _to_rows_tc(xt):
    B, C, N = xt.shape

    def body(x_ref, o_ref):
        o_ref[0] = jnp.transpose(x_ref[0], (1, 0))

    rows = pl.pallas_call(
        body,
        grid=(B,),
        in_specs=[pl.BlockSpec((1, C, N), lambda b: (b, 0, 0))],
        out_specs=pl.BlockSpec((1, N, C), lambda b: (b, 0, 0)),
        out_shape=jax.ShapeDtypeStruct((B, N, C), jnp.float32),
    )(xt)
    return rows.reshape(B * N, C)


def _conv_stats_tc(xt, xm_rows, Wg, b2, BN):
    B, C, N = xt.shape
    G, opg, cpg = Wg.shape
    out_ch = G * opg
    NB = N // BN

    def body(x_ref, xm_ref, w_ref, b_ref, y_ref, s_ref):
        step = pl.program_id(0) * pl.num_programs(1) + pl.program_id(1)
        xblk = x_ref[0]
        xmblk = xm_ref[...]
        parts = []
        for g in range(G):
            w = w_ref[g]
            if g * cpg < C:
                h = xblk[g * cpg:(g + 1) * cpg, :]
                yg = lax.dot_general(
                    w, h, (((1,), (0,)), ((), ())),
                    preferred_element_type=jnp.float32,
                    precision=lax.Precision.HIGHEST,
                )
            else:
                c0 = g * cpg - C
                h = xmblk[:, c0:c0 + cpg]
                yg = lax.dot_general(
                    w, h, (((1,), (1,)), ((), ())),
                    preferred_element_type=jnp.float32,
                    precision=lax.Precision.HIGHEST,
                )
            parts.append(yg)
        y = jnp.concatenate(parts, axis=0) + b_ref[...]
        y_ref[0] = y

        @pl.when(step == 0)
        def _():
            s_ref[...] = jnp.zeros_like(s_ref)

        s_ref[:, 0:1] += jnp.sum(y, axis=1, keepdims=True)
        s_ref[:, 1:2] += jnp.sum(y * y, axis=1, keepdims=True)

    return pl.pallas_call(
        body,
        grid=(B, NB),
        in_specs=[
            pl.BlockSpec((1, C, BN), lambda bi, ni: (bi, 0, ni)),
            pl.BlockSpec((BN, C), lambda bi, ni: (bi * (N // BN) + ni, 0)),
            pl.BlockSpec((G, opg, cpg), lambda bi, ni: (0, 0, 0)),
            pl.BlockSpec((out_ch, 1), lambda bi, ni: (0, 0)),
        ],
        out_specs=[
            pl.BlockSpec((1, out_ch, BN), lambda bi, ni: (bi, 0, ni)),
            pl.BlockSpec((out_ch, 2), lambda bi, ni: (0, 0)),
        ],
        out_shape=[
            jax.ShapeDtypeStruct((B, out_ch, N), jnp.float32),
            jax.ShapeDtypeStruct((out_ch, 2), jnp.float32),
        ],
    )(xt, xm_rows, Wg, b2)


def _bn_gelu_tc(y, sums, g2, be2, BN):
    B, out_ch, N = y.shape
    NB = N // BN
    count = float(B * N)

    def body(y_ref, s_ref, g_ref, be_ref, o_ref):
        mean = s_ref[:, 0:1] * (1.0 / count)
        var = s_ref[:, 1:2] * (1.0 / count) - mean * mean
        inv = lax.rsqrt(var + 1e-5)
        scale = g_ref[...] * inv
        shift = be_ref[...] - mean * scale
        z = y_ref[0] * scale + shift
        o_ref[0] = 0.5 * z * (1.0 + lax.erf(z * 0.7071067811865475))

    return pl.pallas_call(
        body,
        grid=(B, NB),
        in_specs=[
            pl.BlockSpec((1, out_ch, BN), lambda bi, ni: (bi, 0, ni)),
            pl.BlockSpec((out_ch, 2), lambda bi, ni: (0, 0)),
            pl.BlockSpec((out_ch, 1), lambda bi, ni: (0, 0)),
            pl.BlockSpec((out_ch, 1), lambda bi, ni: (0, 0)),
        ],
        out_specs=pl.BlockSpec((1, out_ch, BN), lambda bi, ni: (bi, 0, ni)),
        out_shape=jax.ShapeDtypeStruct((B, out_ch, N), jnp.float32),
    )(y, sums, g2, be2)


@jax.jit
def kernel(x, edge_index, W, b, gamma, beta):
    B, C, N, _ = x.shape
    K = edge_index.shape[-1]
    S = B * N
    out_ch = W.shape[0]
    G = 4
    cpg = (2 * C) // G
    opg = out_ch // G

    xt = x[:, :, :, 0]
    x_rows = _to_rows_tc(xt)
    offs = (jnp.arange(B, dtype=jnp.int32) * N)[None, :, None, None]
    eg = edge_index + offs

    S_pad = ((S + 32 * _WIN - 1) // (32 * _WIN)) * (32 * _WIN)
    npad = S_pad - S
    pad = jnp.zeros((npad * K,), dtype=jnp.int32)
    i_src = jnp.concatenate([eg[0].reshape(S * K), pad]).reshape(-1, 1, _WIN * K)
    i_dst = jnp.concatenate([eg[1].reshape(S * K), pad]).reshape(-1, 1, _WIN * K)

    xmax_rows = _mr_aggregate_sc(x_rows, i_src, i_dst, K)

    Wg = W[:, :, 0, 0].reshape(G, opg, cpg)
    BN = N
    y, sums = _conv_stats_tc(xt, xmax_rows, Wg, b.reshape(out_ch, 1), BN)
    out = _bn_gelu_tc(y, sums, gamma.reshape(out_ch, 1),
                      beta.reshape(out_ch, 1), BN)
    return out.reshape(B, out_ch, N, 1)
